# baseline (speedup 1.0000x reference)
import functools

import jax
import jax.numpy as jnp
from jax import lax
from jax.experimental import pallas as pl
from jax.experimental.pallas import tpu as pltpu
from jax.experimental.pallas import tpu_sc as plsc

NUM_CORES = 2
NUM_SUBCORES = 16
LANES = 16
NUM_WORKERS = NUM_CORES * NUM_SUBCORES

BATCH = 16384
FACTORS = 64
NUM_USERS = 1000000
PADDED = 2 * FACTORS
B_PER_W = BATCH // NUM_WORKERS
GLOBAL_MEAN = 3.5


@functools.partial(
    pl.kernel,
    out_type=jax.ShapeDtypeStruct((BATCH,), jnp.float32),
    mesh=plsc.VectorSubcoreMesh(core_axis_name="c", subcore_axis_name="s"),
    compiler_params=pltpu.CompilerParams(needs_layout_passes=False,
                                         use_tc_tiling_on_sc=True),
    scratch_types=[
        pltpu.VMEM((B_PER_W,), jnp.int32),
        pltpu.VMEM((B_PER_W,), jnp.int32),
        pltpu.VMEM((B_PER_W,), jnp.int32),
        pltpu.VMEM((B_PER_W,), jnp.int32),
        pltpu.VMEM((FACTORS * B_PER_W,), jnp.float32),
        pltpu.VMEM((B_PER_W, PADDED), jnp.float32),
        pltpu.VMEM((B_PER_W,), jnp.float32),
        pltpu.VMEM((B_PER_W,), jnp.float32),
        pltpu.VMEM((B_PER_W,), jnp.float32),
        pltpu.SemaphoreType.DMA,
        pltpu.SemaphoreType.DMA,
        pltpu.SemaphoreType.DMA,
        pltpu.SemaphoreType.DMA,
    ],
)
def _mf_sc_kernel(users_hbm, items_hbm, uembT_hbm, iemb_hbm, ubias_hbm,
                  ibias_hbm, out_hbm, uidx_v, iidx_v, irow_v, ioff_v,
                  uvals_v, irows_v, ub_v, ib_v, out_v,
                  sem_u, sem_i, sem_ub, sem_ib):
    wid = lax.axis_index("s") * NUM_CORES + lax.axis_index("c")
    base = wid * B_PER_W

    pltpu.sync_copy(users_hbm.at[pl.ds(base, B_PER_W)], uidx_v)
    pltpu.sync_copy(items_hbm.at[pl.ds(base, B_PER_W)], iidx_v)

    cub = pltpu.async_copy(ubias_hbm.at[uidx_v], ub_v, sem_ub)
    cib = pltpu.async_copy(ibias_hbm.at[iidx_v], ib_v, sem_ib)

    def prep_body(j, carry):
        sl = pl.ds(j * LANES, LANES)
        i = iidx_v[sl]
        irow_v[sl] = lax.shift_right_logical(i, 1)
        ioff_v[sl] = lax.shift_left(jnp.bitwise_and(i, 1), 6)
        return carry

    lax.fori_loop(0, B_PER_W // LANES, prep_body, 0)

    ci = pltpu.async_copy(iemb_hbm.at[irow_v], irows_v, sem_i)

    def fire_body(f, carry):
        src = uembT_hbm.at[pl.ds(f * NUM_USERS, NUM_USERS)].at[uidx_v]
        dst = uvals_v.at[pl.ds(f * B_PER_W, B_PER_W)]
        pltpu.async_copy(src, dst, sem_u)
        return carry

    lax.fori_loop(0, FACTORS, fire_body, 0)

    pltpu.make_async_copy(
        uembT_hbm.at[pl.ds(0, FACTORS * B_PER_W)], uvals_v, sem_u
    ).wait()
    ci.wait()
    cub.wait()
    cib.wait()

    lane = lax.iota(jnp.int32, LANES)

    def group_body(g, carry):
        gbase = g * LANES
        rows = gbase + lane
        ioff = ioff_v[pl.ds(gbase, LANES)]
        acc = (ub_v[pl.ds(gbase, LANES)] + ib_v[pl.ds(gbase, LANES)]
               + GLOBAL_MEAN)
        for f in range(FACTORS):
            u = uvals_v[pl.ds(f * B_PER_W + gbase, LANES)]
            v = plsc.load_gather(irows_v, [rows, ioff + f])
            acc = acc + u * v
        out_v[pl.ds(gbase, LANES)] = acc
        return carry

    lax.fori_loop(0, B_PER_W // LANES, group_body, 0)

    pltpu.sync_copy(out_v, out_hbm.at[pl.ds(base, B_PER_W)])


def kernel(users, items, user_emb, item_emb, user_bias, item_bias):
    uembT = user_emb.T.reshape(-1)
    iemb2 = item_emb.reshape(-1, PADDED)
    return _mf_sc_kernel(users.astype(jnp.int32), items.astype(jnp.int32),
                         uembT, iemb2,
                         user_bias.reshape(-1), item_bias.reshape(-1))


# speedup vs baseline: 7.3789x; 7.3789x over previous
import functools

import jax
import jax.numpy as jnp
from jax import lax
from jax.experimental import pallas as pl
from jax.experimental.pallas import tpu as pltpu
from jax.experimental.pallas import tpu_sc as plsc

NUM_CORES = 2
NUM_SUBCORES = 16
LANES = 16
NUM_WORKERS = NUM_CORES * NUM_SUBCORES

BATCH = 16384
FACTORS = 64
PADDED = 2 * FACTORS
B_PER_W = BATCH // NUM_WORKERS
CHUNK = 128
N_CHUNKS = B_PER_W // CHUNK
GLOBAL_MEAN = 3.5


@functools.partial(
    pl.kernel,
    out_type=jax.ShapeDtypeStruct((BATCH,), jnp.float32),
    mesh=plsc.VectorSubcoreMesh(core_axis_name="c", subcore_axis_name="s"),
    compiler_params=pltpu.CompilerParams(needs_layout_passes=False,
                                         use_tc_tiling_on_sc=True),
    scratch_types=[
        pltpu.VMEM((B_PER_W,), jnp.int32),
        pltpu.VMEM((B_PER_W,), jnp.int32),
        pltpu.VMEM((B_PER_W,), jnp.int32),
        pltpu.VMEM((B_PER_W,), jnp.int32),
        pltpu.VMEM((B_PER_W,), jnp.int32),
        pltpu.VMEM((B_PER_W,), jnp.int32),
        pltpu.VMEM((CHUNK, PADDED), jnp.float32),
        pltpu.VMEM((CHUNK, PADDED), jnp.float32),
        pltpu.VMEM((CHUNK, PADDED), jnp.float32),
        pltpu.VMEM((CHUNK, PADDED), jnp.float32),
        pltpu.VMEM((B_PER_W,), jnp.float32),
        pltpu.VMEM((B_PER_W,), jnp.float32),
        pltpu.VMEM((B_PER_W,), jnp.float32),
        pltpu.SemaphoreType.DMA,
        pltpu.SemaphoreType.DMA,
        pltpu.SemaphoreType.DMA,
        pltpu.SemaphoreType.DMA,
        pltpu.SemaphoreType.DMA,
        pltpu.SemaphoreType.DMA,
    ],
)
def _mf_sc_kernel(users_hbm, items_hbm, uemb_hbm, iemb_hbm, ubias_hbm,
                  ibias_hbm, out_hbm, uidx_v, iidx_v, urow_v, uoff_v,
                  irow_v, ioff_v, ubufA, ubufB, ibufA, ibufB,
                  ub_v, ib_v, out_v,
                  sem_uA, sem_uB, sem_iA, sem_iB, sem_ub, sem_ib):
    wid = lax.axis_index("s") * NUM_CORES + lax.axis_index("c")
    base = wid * B_PER_W

    pltpu.sync_copy(users_hbm.at[pl.ds(base, B_PER_W)], uidx_v)
    pltpu.sync_copy(items_hbm.at[pl.ds(base, B_PER_W)], iidx_v)

    cub = pltpu.async_copy(ubias_hbm.at[uidx_v], ub_v, sem_ub)
    cib = pltpu.async_copy(ibias_hbm.at[iidx_v], ib_v, sem_ib)

    def prep_body(j, carry):
        sl = pl.ds(j * LANES, LANES)
        u = uidx_v[sl]
        urow_v[sl] = lax.shift_right_logical(u, 1)
        uoff_v[sl] = lax.shift_left(jnp.bitwise_and(u, 1), 6)
        i = iidx_v[sl]
        irow_v[sl] = lax.shift_right_logical(i, 1)
        ioff_v[sl] = lax.shift_left(jnp.bitwise_and(i, 1), 6)
        return carry

    lax.fori_loop(0, B_PER_W // LANES, prep_body, 0)

    ubufs = (ubufA, ubufB)
    ibufs = (ibufA, ibufB)
    usems = (sem_uA, sem_uB)
    isems = (sem_iA, sem_iB)

    def fire(c):
        p = c % 2
        sl = pl.ds(c * CHUNK, CHUNK)
        cu = pltpu.async_copy(uemb_hbm.at[urow_v.at[sl]], ubufs[p], usems[p])
        ci = pltpu.async_copy(iemb_hbm.at[irow_v.at[sl]], ibufs[p], isems[p])
        return cu, ci

    pend = fire(0)
    cub.wait()
    cib.wait()

    lane = lax.iota(jnp.int32, LANES)

    for c in range(N_CHUNKS):
        p = c % 2
        cu, ci = pend
        if c + 1 < N_CHUNKS:
            nxt = fire(c + 1)
        cu.wait()
        ci.wait()
        if c + 1 < N_CHUNKS:
            pend = nxt
        ubuf = ubufs[p]
        ibuf = ibufs[p]
        cbase = c * CHUNK

        def group_body(g, carry, ubuf=ubuf, ibuf=ibuf, cbase=cbase):
            gbase = cbase + g * LANES
            rows = g * LANES + lane
            uoff = uoff_v[pl.ds(gbase, LANES)]
            ioff = ioff_v[pl.ds(gbase, LANES)]
            acc = (ub_v[pl.ds(gbase, LANES)] + ib_v[pl.ds(gbase, LANES)]
                   + GLOBAL_MEAN)
            for f in range(FACTORS):
                u = plsc.load_gather(ubuf, [rows, uoff + f])
                v = plsc.load_gather(ibuf, [rows, ioff + f])
                acc = acc + u * v
            out_v[pl.ds(gbase, LANES)] = acc
            return carry

        lax.fori_loop(0, CHUNK // LANES, group_body, 0)

    pltpu.sync_copy(out_v, out_hbm.at[pl.ds(base, B_PER_W)])


def kernel(users, items, user_emb, item_emb, user_bias, item_bias):
    uemb2 = user_emb.reshape(-1, PADDED)
    iemb2 = item_emb.reshape(-1, PADDED)
    return _mf_sc_kernel(users.astype(jnp.int32), items.astype(jnp.int32),
                         uemb2, iemb2,
                         user_bias.reshape(-1), item_bias.reshape(-1))


# speedup vs baseline: 7.4953x; 1.0158x over previous
import functools

import jax
import jax.numpy as jnp
from jax import lax
from jax.experimental import pallas as pl
from jax.experimental.pallas import tpu as pltpu
from jax.experimental.pallas import tpu_sc as plsc

NUM_CORES = 2
NUM_SUBCORES = 16
LANES = 16
NUM_WORKERS = NUM_CORES * NUM_SUBCORES

BATCH = 16384
FACTORS = 64
B_PER_W = BATCH // NUM_WORKERS
CHUNK = 256
N_CHUNKS = B_PER_W // CHUNK
GLOBAL_MEAN = 3.5


@functools.partial(
    pl.kernel,
    out_type=jax.ShapeDtypeStruct((BATCH,), jnp.float32),
    mesh=plsc.VectorSubcoreMesh(core_axis_name="c", subcore_axis_name="s"),
    compiler_params=pltpu.CompilerParams(needs_layout_passes=False,
                                         use_tc_tiling_on_sc=False),
    scratch_types=[
        pltpu.VMEM((B_PER_W,), jnp.int32),
        pltpu.VMEM((B_PER_W,), jnp.int32),
        pltpu.VMEM((CHUNK, FACTORS), jnp.float32),
        pltpu.VMEM((CHUNK, FACTORS), jnp.float32),
        pltpu.VMEM((CHUNK, FACTORS), jnp.float32),
        pltpu.VMEM((CHUNK, FACTORS), jnp.float32),
        pltpu.VMEM((B_PER_W,), jnp.float32),
        pltpu.VMEM((B_PER_W,), jnp.float32),
        pltpu.VMEM((B_PER_W,), jnp.float32),
        pltpu.SemaphoreType.DMA,
        pltpu.SemaphoreType.DMA,
        pltpu.SemaphoreType.DMA,
        pltpu.SemaphoreType.DMA,
        pltpu.SemaphoreType.DMA,
        pltpu.SemaphoreType.DMA,
    ],
)
def _mf_sc_kernel(users_hbm, items_hbm, uemb_hbm, iemb_hbm, ubias_hbm,
                  ibias_hbm, out_hbm, uidx_v, iidx_v,
                  ubufA, ubufB, ibufA, ibufB, ub_v, ib_v, out_v,
                  sem_uA, sem_uB, sem_iA, sem_iB, sem_ub, sem_ib):
    wid = lax.axis_index("s") * NUM_CORES + lax.axis_index("c")
    base = wid * B_PER_W

    pltpu.sync_copy(users_hbm.at[pl.ds(base, B_PER_W)], uidx_v)
    pltpu.sync_copy(items_hbm.at[pl.ds(base, B_PER_W)], iidx_v)

    cub = pltpu.async_copy(ubias_hbm.at[uidx_v], ub_v, sem_ub)
    cib = pltpu.async_copy(ibias_hbm.at[iidx_v], ib_v, sem_ib)

    ubufs = (ubufA, ubufB)
    ibufs = (ibufA, ibufB)
    usems = (sem_uA, sem_uB)
    isems = (sem_iA, sem_iB)

    def fire(c):
        p = c % 2
        sl = pl.ds(c * CHUNK, CHUNK)
        cu = pltpu.async_copy(uemb_hbm.at[uidx_v.at[sl]], ubufs[p], usems[p])
        ci = pltpu.async_copy(iemb_hbm.at[iidx_v.at[sl]], ibufs[p], isems[p])
        return cu, ci

    pend = fire(0)
    cub.wait()
    cib.wait()

    lane = lax.iota(jnp.int32, LANES)

    for c in range(N_CHUNKS):
        p = c % 2
        cu, ci = pend
        if c + 1 < N_CHUNKS:
            nxt = fire(c + 1)
        cu.wait()
        ci.wait()
        if c + 1 < N_CHUNKS:
            pend = nxt
        ubuf = ubufs[p]
        ibuf = ibufs[p]
        cbase = c * CHUNK

        def group_body(g, carry, ubuf=ubuf, ibuf=ibuf, cbase=cbase):
            gbase = cbase + g * LANES
            rows = g * LANES + lane
            acc = (ub_v[pl.ds(gbase, LANES)] + ib_v[pl.ds(gbase, LANES)]
                   + GLOBAL_MEAN)
            for f in range(FACTORS):
                cols = jnp.full((LANES,), f, jnp.int32)
                u = plsc.load_gather(ubuf, [rows, cols])
                v = plsc.load_gather(ibuf, [rows, cols])
                acc = acc + u * v
            out_v[pl.ds(gbase, LANES)] = acc
            return carry

        lax.fori_loop(0, CHUNK // LANES, group_body, 0)

    pltpu.sync_copy(out_v, out_hbm.at[pl.ds(base, B_PER_W)])


def kernel(users, items, user_emb, item_emb, user_bias, item_bias):
    return _mf_sc_kernel(users.astype(jnp.int32), items.astype(jnp.int32),
                         user_emb, item_emb,
                         user_bias.reshape(-1), item_bias.reshape(-1))


# speedup vs baseline: 10.4643x; 1.3961x over previous
import functools

import jax
import jax.numpy as jnp
from jax import lax
from jax.experimental import pallas as pl
from jax.experimental.pallas import tpu as pltpu
from jax.experimental.pallas import tpu_sc as plsc

NUM_CORES = 2
NUM_SUBCORES = 16
LANES = 16
NUM_WORKERS = NUM_CORES * NUM_SUBCORES

BATCH = 16384
FACTORS = 64
B_PER_W = BATCH // NUM_WORKERS
CHUNK = 128
N_CHUNKS = B_PER_W // CHUNK
GLOBAL_MEAN = 3.5


@functools.partial(
    pl.kernel,
    out_type=jax.ShapeDtypeStruct((BATCH,), jnp.float32),
    mesh=plsc.VectorSubcoreMesh(core_axis_name="c", subcore_axis_name="s"),
    compiler_params=pltpu.CompilerParams(needs_layout_passes=False,
                                         use_tc_tiling_on_sc=True),
    scratch_types=[
        pltpu.VMEM((B_PER_W,), jnp.int32),
        pltpu.VMEM((B_PER_W,), jnp.int32),
        pltpu.VMEM((B_PER_W,), jnp.int32),
        pltpu.VMEM((B_PER_W,), jnp.int32),
        pltpu.VMEM((CHUNK * 2, FACTORS), jnp.float32),
        pltpu.VMEM((CHUNK * 2, FACTORS), jnp.float32),
        pltpu.VMEM((B_PER_W,), jnp.float32),
        pltpu.VMEM((B_PER_W,), jnp.float32),
        pltpu.VMEM((B_PER_W,), jnp.float32),
        pltpu.SemaphoreType.DMA,
        pltpu.SemaphoreType.DMA,
        pltpu.SemaphoreType.DMA,
        pltpu.SemaphoreType.DMA,
    ],
)
def _mf_sc_kernel(users_hbm, items_hbm, uemb_hbm, iemb_hbm, ubias_hbm,
                  ibias_hbm, out_hbm, uidx_v, iidx_v, uhalf_v, ihalf_v,
                  ubuf, ibuf, ub_v, ib_v, out_v,
                  sem_u, sem_i, sem_ub, sem_ib):
    wid = lax.axis_index("s") * NUM_CORES + lax.axis_index("c")
    base = wid * B_PER_W

    pltpu.sync_copy(users_hbm.at[pl.ds(base, B_PER_W)], uidx_v)
    pltpu.sync_copy(items_hbm.at[pl.ds(base, B_PER_W)], iidx_v)

    cub = pltpu.async_copy(ubias_hbm.at[uidx_v], ub_v, sem_ub)
    cib = pltpu.async_copy(ibias_hbm.at[iidx_v], ib_v, sem_ib)

    def prep_body(j, carry):
        sl = pl.ds(j * LANES, LANES)
        uhalf_v[sl] = jnp.bitwise_and(uidx_v[sl], 1)
        ihalf_v[sl] = jnp.bitwise_and(iidx_v[sl], 1)
        return carry

    lax.fori_loop(0, B_PER_W // LANES, prep_body, 0)

    lane16 = lax.iota(jnp.int32, LANES)

    def uscal(j):
        g = lax.shift_right_logical(j, 4)
        chunk = uidx_v[pl.ds(g * LANES, LANES)]
        return jnp.max(jnp.where(lane16 == jnp.bitwise_and(j, 15), chunk, 0))

    def iscal(j):
        g = lax.shift_right_logical(j, 4)
        chunk = iidx_v[pl.ds(g * LANES, LANES)]
        return jnp.max(jnp.where(lane16 == jnp.bitwise_and(j, 15), chunk, 0))

    def upair(j):
        pr2 = pl.multiple_of(jnp.bitwise_and(uscal(j), -2), 2)
        return uemb_hbm.at[pl.ds(pr2, 2), pl.ds(0, FACTORS)]

    def ipair(j):
        pr2 = pl.multiple_of(jnp.bitwise_and(iscal(j), -2), 2)
        return iemb_hbm.at[pl.ds(pr2, 2), pl.ds(0, FACTORS)]

    def udst(j):
        return ubuf.at[pl.ds(pl.multiple_of(j * 2, 2), 2), pl.ds(0, FACTORS)]

    def idst(j):
        return ibuf.at[pl.ds(pl.multiple_of(j * 2, 2), 2), pl.ds(0, FACTORS)]

    cub.wait()
    cib.wait()

    lane = lax.iota(jnp.int32, LANES)

    for c in range(N_CHUNKS):
        cb = c * CHUNK

        def fire_body(j, carry, cb=cb):
            pltpu.async_copy(upair(cb + j), udst(j), sem_u)
            pltpu.async_copy(ipair(cb + j), idst(j), sem_i)
            return carry

        lax.fori_loop(0, CHUNK, fire_body, 0)

        pltpu.make_async_copy(
            uemb_hbm.at[pl.ds(0, CHUNK * 2), pl.ds(0, FACTORS)], ubuf,
            sem_u).wait()
        pltpu.make_async_copy(
            iemb_hbm.at[pl.ds(0, CHUNK * 2), pl.ds(0, FACTORS)], ibuf,
            sem_i).wait()

        def group_body(g, carry, cb=cb):
            gbase = cb + g * LANES
            rows = g * LANES + lane
            uhalf = uhalf_v[pl.ds(gbase, LANES)]
            ihalf = ihalf_v[pl.ds(gbase, LANES)]
            acc = (ub_v[pl.ds(gbase, LANES)] + ib_v[pl.ds(gbase, LANES)]
                   + GLOBAL_MEAN)
            for f in range(FACTORS):
                cols = jnp.full((LANES,), f, jnp.int32)
                u = plsc.load_gather(ubuf, [rows * 2 + uhalf, cols])
                v = plsc.load_gather(ibuf, [rows * 2 + ihalf, cols])
                acc = acc + u * v
            out_v[pl.ds(gbase, LANES)] = acc
            return carry

        lax.fori_loop(0, CHUNK // LANES, group_body, 0)

    pltpu.sync_copy(out_v, out_hbm.at[pl.ds(base, B_PER_W)])


def kernel(users, items, user_emb, item_emb, user_bias, item_bias):
    return _mf_sc_kernel(users.astype(jnp.int32), items.astype(jnp.int32),
                         user_emb, item_emb,
                         user_bias.reshape(-1), item_bias.reshape(-1))


# speedup vs baseline: 10.6463x; 1.0174x over previous
import functools

import jax
import jax.numpy as jnp
from jax import lax
from jax.experimental import pallas as pl
from jax.experimental.pallas import tpu as pltpu
from jax.experimental.pallas import tpu_sc as plsc

NUM_CORES = 2
NUM_SUBCORES = 16
LANES = 16
NUM_WORKERS = NUM_CORES * NUM_SUBCORES

BATCH = 16384
FACTORS = 64
B_PER_W = BATCH // NUM_WORKERS
CHUNK = 128
N_CHUNKS = B_PER_W // CHUNK
GLOBAL_MEAN = 3.5


@functools.partial(
    pl.kernel,
    out_type=jax.ShapeDtypeStruct((BATCH,), jnp.float32),
    mesh=plsc.VectorSubcoreMesh(core_axis_name="c", subcore_axis_name="s"),
    compiler_params=pltpu.CompilerParams(needs_layout_passes=False,
                                         use_tc_tiling_on_sc=True),
    scratch_types=[
        pltpu.VMEM((B_PER_W,), jnp.int32),
        pltpu.VMEM((B_PER_W,), jnp.int32),
        pltpu.VMEM((B_PER_W,), jnp.int32),
        pltpu.VMEM((B_PER_W,), jnp.int32),
        pltpu.VMEM((CHUNK * 2, FACTORS), jnp.float32),
        pltpu.VMEM((CHUNK * 2, FACTORS), jnp.float32),
        pltpu.VMEM((B_PER_W,), jnp.float32),
        pltpu.VMEM((B_PER_W,), jnp.float32),
        pltpu.VMEM((B_PER_W,), jnp.float32),
        pltpu.SemaphoreType.DMA,
        pltpu.SemaphoreType.DMA,
        pltpu.SemaphoreType.DMA,
        pltpu.SemaphoreType.DMA,
    ],
)
def _mf_sc_kernel(users_hbm, items_hbm, uemb_hbm, iemb_hbm, ubias_hbm,
                  ibias_hbm, out_hbm, uidx_v, iidx_v, uhalf_v, ihalf_v,
                  ubuf, ibuf, ub_v, ib_v, out_v,
                  sem_u, sem_i, sem_ub, sem_ib):
    wid = lax.axis_index("s") * NUM_CORES + lax.axis_index("c")
    base = wid * B_PER_W

    pltpu.sync_copy(users_hbm.at[pl.ds(base, B_PER_W)], uidx_v)
    pltpu.sync_copy(items_hbm.at[pl.ds(base, B_PER_W)], iidx_v)

    cub = pltpu.async_copy(ubias_hbm.at[uidx_v], ub_v, sem_ub)
    cib = pltpu.async_copy(ibias_hbm.at[iidx_v], ib_v, sem_ib)

    def prep_body(j, carry):
        sl = pl.ds(j * LANES, LANES)
        uhalf_v[sl] = jnp.bitwise_and(uidx_v[sl], 1)
        ihalf_v[sl] = jnp.bitwise_and(iidx_v[sl], 1)
        return carry

    lax.fori_loop(0, B_PER_W // LANES, prep_body, 0)

    lane16 = lax.iota(jnp.int32, LANES)

    def uscal(j):
        g = lax.shift_right_logical(j, 4)
        chunk = uidx_v[pl.ds(g * LANES, LANES)]
        return jnp.max(jnp.where(lane16 == jnp.bitwise_and(j, 15), chunk, 0))

    def iscal(j):
        g = lax.shift_right_logical(j, 4)
        chunk = iidx_v[pl.ds(g * LANES, LANES)]
        return jnp.max(jnp.where(lane16 == jnp.bitwise_and(j, 15), chunk, 0))

    def upair(j):
        pr2 = pl.multiple_of(jnp.bitwise_and(uscal(j), -2), 2)
        return uemb_hbm.at[pl.ds(pr2, 2), pl.ds(0, FACTORS)]

    def ipair(j):
        pr2 = pl.multiple_of(jnp.bitwise_and(iscal(j), -2), 2)
        return iemb_hbm.at[pl.ds(pr2, 2), pl.ds(0, FACTORS)]

    def udst(j):
        return ubuf.at[pl.ds(pl.multiple_of(j * 2, 2), 2), pl.ds(0, FACTORS)]

    def idst(j):
        return ibuf.at[pl.ds(pl.multiple_of(j * 2, 2), 2), pl.ds(0, FACTORS)]

    cub.wait()
    cib.wait()

    lane = lax.iota(jnp.int32, LANES)

    for c in range(N_CHUNKS):
        cb = c * CHUNK

        def fire_group(g, carry, cb=cb):
            gb = g * LANES
            uchunk = uidx_v[pl.ds(cb + gb, LANES)]
            ichunk = iidx_v[pl.ds(cb + gb, LANES)]
            for rr in range(LANES):
                us = jnp.max(jnp.where(lane16 == rr, uchunk, 0))
                vs = jnp.max(jnp.where(lane16 == rr, ichunk, 0))
                upr = pl.multiple_of(jnp.bitwise_and(us, -2), 2)
                ipr = pl.multiple_of(jnp.bitwise_and(vs, -2), 2)
                pltpu.async_copy(
                    uemb_hbm.at[pl.ds(upr, 2), pl.ds(0, FACTORS)],
                    udst(gb + rr), sem_u)
                pltpu.async_copy(
                    iemb_hbm.at[pl.ds(ipr, 2), pl.ds(0, FACTORS)],
                    idst(gb + rr), sem_i)
            return carry

        lax.fori_loop(0, CHUNK // LANES, fire_group, 0)

        pltpu.make_async_copy(
            uemb_hbm.at[pl.ds(0, CHUNK * 2), pl.ds(0, FACTORS)], ubuf,
            sem_u).wait()
        pltpu.make_async_copy(
            iemb_hbm.at[pl.ds(0, CHUNK * 2), pl.ds(0, FACTORS)], ibuf,
            sem_i).wait()

        def group_body(g, carry, cb=cb):
            gbase = cb + g * LANES
            rows = g * LANES + lane
            uhalf = uhalf_v[pl.ds(gbase, LANES)]
            ihalf = ihalf_v[pl.ds(gbase, LANES)]
            acc = (ub_v[pl.ds(gbase, LANES)] + ib_v[pl.ds(gbase, LANES)]
                   + GLOBAL_MEAN)
            for f in range(FACTORS):
                cols = jnp.full((LANES,), f, jnp.int32)
                u = plsc.load_gather(ubuf, [rows * 2 + uhalf, cols])
                v = plsc.load_gather(ibuf, [rows * 2 + ihalf, cols])
                acc = acc + u * v
            out_v[pl.ds(gbase, LANES)] = acc
            return carry

        lax.fori_loop(0, CHUNK // LANES, group_body, 0)

    pltpu.sync_copy(out_v, out_hbm.at[pl.ds(base, B_PER_W)])


def kernel(users, items, user_emb, item_emb, user_bias, item_bias):
    return _mf_sc_kernel(users.astype(jnp.int32), items.astype(jnp.int32),
                         user_emb, item_emb,
                         user_bias.reshape(-1), item_bias.reshape(-1))
